# Initial kernel scaffold; baseline (speedup 1.0000x reference)
import jax
import jax.numpy as jnp
from jax.experimental import pallas as pl


def kernel(input, labels):
    raise NotImplementedError("write your pallas kernel here")


# speedup vs baseline: 2.2924x; 2.2924x over previous
import functools

import jax
import jax.numpy as jnp
from jax.experimental import pallas as pl
from jax.experimental.pallas import tpu as pltpu

_S = 32.0
_M = 0.5


def _loss_kernel(labels_ref, x_ref, out_ref, m_ref, s_ref, t_ref, *, cb, c_total,
                 b_total, s_scale, margin):
    r = pl.program_id(0)
    c = pl.program_id(1)
    nr = pl.num_programs(0)
    nc = pl.num_programs(1)

    @pl.when(c == 0)
    def _init():
        m_ref[...] = jnp.full_like(m_ref, -jnp.inf)
        s_ref[...] = jnp.zeros_like(s_ref)
        t_ref[...] = jnp.zeros_like(t_ref)

    y = x_ref[...] * s_scale
    lab = labels_ref[0, 0, :]
    cols = c * cb + jax.lax.broadcasted_iota(jnp.int32, y.shape, 1)
    is_lab = cols == lab[:, None]
    y = jnp.where(is_lab, y - s_scale * margin, y)
    y = jnp.where(cols < c_total, y, -jnp.inf)

    t_ref[...] += jnp.sum(jnp.where(is_lab, y, 0.0), axis=1, keepdims=True)

    m_prev = m_ref[...]
    m_new = jnp.maximum(m_prev, jnp.max(y, axis=1, keepdims=True))
    s_ref[...] = s_ref[...] * jnp.exp(m_prev - m_new) + jnp.sum(
        jnp.exp(y - m_new), axis=1, keepdims=True)
    m_ref[...] = m_new

    @pl.when(c == nc - 1)
    def _finish():
        lse = m_ref[...] + jnp.log(s_ref[...])
        part = jnp.sum(lse - t_ref[...])
        prev = jnp.where(r == 0, 0.0, out_ref[0, 0])
        tot = prev + part
        out_ref[0, 0] = jnp.where(r == nr - 1, tot / b_total, tot)


def kernel(input, labels):
    b, c_total = input.shape
    r_blk = min(256, b)
    cb = 2048
    rb = b // r_blk
    nc = pl.cdiv(c_total, cb)

    labels_r = labels.astype(jnp.int32).reshape(rb, 1, r_blk)

    out = pl.pallas_call(
        functools.partial(_loss_kernel, cb=cb, c_total=c_total, b_total=b,
                          s_scale=_S, margin=_M),
        grid=(rb, nc),
        in_specs=[
            pl.BlockSpec((1, 1, r_blk), lambda r, c: (r, 0, 0)),
            pl.BlockSpec((r_blk, cb), lambda r, c: (r, c)),
        ],
        out_specs=pl.BlockSpec((1, 1), lambda r, c: (0, 0),
                               memory_space=pltpu.SMEM),
        out_shape=jax.ShapeDtypeStruct((1, 1), jnp.float32),
        scratch_shapes=[
            pltpu.VMEM((r_blk, 1), jnp.float32),
            pltpu.VMEM((r_blk, 1), jnp.float32),
            pltpu.VMEM((r_blk, 1), jnp.float32),
        ],
        compiler_params=pltpu.CompilerParams(
            dimension_semantics=("arbitrary", "arbitrary")),
    )(labels_r, input)
    return out[0, 0]
